# Initial kernel scaffold; baseline (speedup 1.0000x reference)
import jax
import jax.numpy as jnp
from jax.experimental import pallas as pl


def kernel(center, normal, feature, offset, W0, b0, gamma0, beta0, W1, b1, gamma1, beta1, W2, b2, gamma2, beta2):
    raise NotImplementedError("write your pallas kernel here")


# speedup vs baseline: 3.9241x; 3.9241x over previous
import functools
import numpy as np
import jax
import jax.numpy as jnp
from jax import lax
from jax.experimental import pallas as pl
from jax.experimental.pallas import tpu as pltpu
from jax.experimental.pallas import tpu_sc as plsc

N = 10000
NS = 32
NPAD = 10240
NQPAD = 10112
QB = 128
MB = 6400
QMB = 200
M = N * NS
TBL_D = 128
BIG = float(1e30)
BIGI = int(2**30)
CNT = float(M)
EPS = float(1e-5)


def _prep_body(cpad_ref, nrm_ref, feat_ref, w0_ref, psqc_ref, tbl_ref):
    c = cpad_ref[...]
    x, y, z = c[:, 0:1], c[:, 1:2], c[:, 2:3]
    psqc_ref[...] = (x * x + y * y) + z * z
    w0f = w0_ref[:, 9:137]
    pf = lax.dot_general(feat_ref[...], w0f, (((1,), (1,)), ((), ())),
                         preferred_element_type=jnp.float32)
    tbl_ref[...] = jnp.concatenate(
        [pf, c[:N], nrm_ref[...], jnp.zeros((N, TBL_D - 70), jnp.float32)],
        axis=1)


def _prep(cpad, normal, feature, w0):
    return pl.pallas_call(
        _prep_body,
        out_shape=(
            jax.ShapeDtypeStruct((NPAD, 1), jnp.float32),
            jax.ShapeDtypeStruct((N, TBL_D), jnp.float32),
        ),
    )(cpad, normal, feature, w0)


def _knn_body(qp_ref, pp_ref, psqr_ref, idx_ref):
    q = qp_ref[...]
    p = pp_ref[...]
    qx, qy, qz = q[:, 0:1], q[:, 1:2], q[:, 2:3]
    qsq = (qx * qx + qy * qy) + qz * qz
    dot2 = lax.dot_general(q.astype(jnp.bfloat16), p.astype(jnp.bfloat16),
                           (((1,), (1,)), ((), ())),
                           preferred_element_type=jnp.float32)
    d = qsq + psqr_ref[0:1, :] - 2.0 * dot2
    d3 = d.reshape(QB, 10, 1024)
    i10 = lax.broadcasted_iota(jnp.int32, (QB, 10, 1024), 1)
    i1024 = lax.broadcasted_iota(jnp.int32, (QB, 10, 1024), 2)
    col3 = i10 * 1024 + i1024
    vals, idxs = [], []
    for _ in range(4):
        m = jnp.min(d3, axis=1)
        eq = d3 == m[:, None, :]
        ai = jnp.min(jnp.where(eq, col3, BIGI), axis=1)
        d3 = jnp.where(eq & (col3 == ai[:, None, :]), BIG, d3)
        vals.append(m)
        idxs.append(ai)
    v4 = jnp.stack(vals, axis=1)
    c4 = jnp.stack(idxs, axis=1)
    v2 = v4.reshape(QB, 32, 128)
    c2 = c4.reshape(QB, 32, 128)
    vals2, idxs2 = [], []
    for _ in range(8):
        m = jnp.min(v2, axis=1)
        eq = v2 == m[:, None, :]
        ai = jnp.min(jnp.where(eq, c2, BIGI), axis=1)
        v2 = jnp.where(eq & (c2 == ai[:, None, :]), BIG, v2)
        vals2.append(m)
        idxs2.append(ai)
    cand = jnp.stack(vals2, axis=1).reshape(QB, 1024)
    cidx = jnp.stack(idxs2, axis=1).reshape(QB, 1024)
    cols = []
    for _ in range(NS):
        m = jnp.min(cand, axis=1)
        eq = cand == m[:, None]
        ai = jnp.min(jnp.where(eq, cidx, BIGI), axis=1)
        cand = jnp.where(eq & (cidx == ai[:, None]), BIG, cand)
        cols.append(ai)
    idx_ref[...] = jnp.stack(cols, axis=1)


def _knn(qp, pp, psqr):
    return pl.pallas_call(
        _knn_body,
        grid=(NQPAD // QB,),
        in_specs=[
            pl.BlockSpec((QB, 3), lambda i: (i, 0)),
            pl.BlockSpec((NPAD, 3), lambda i: (0, 0)),
            pl.BlockSpec((1, NPAD), lambda i: (0, 0)),
        ],
        out_specs=pl.BlockSpec((QB, NS), lambda i: (i, 0)),
        out_shape=jax.ShapeDtypeStruct((NQPAD, NS), jnp.int32),
    )(qp, pp, psqr)


def _make_sc_gather():
    info = plsc.get_sparse_core_info()
    nw = info.num_cores * info.num_subcores
    b_per_w = M // nw
    chunk = 400
    nchunk = b_per_w // chunk
    mesh = plsc.VectorSubcoreMesh(core_axis_name="c", subcore_axis_name="s")

    @functools.partial(
        pl.kernel, mesh=mesh,
        out_type=jax.ShapeDtypeStruct((M, TBL_D), jnp.float32),
        scratch_types=[
            pltpu.VMEM((chunk,), jnp.int32),
            pltpu.VMEM((chunk, TBL_D), jnp.float32),
            pltpu.SemaphoreType.DMA,
        ],
    )
    def k(tbl_hbm, idx_hbm, out_hbm, idx_v, rows_v, sem):
        wid = lax.axis_index("s") * info.num_cores + lax.axis_index("c")
        base = wid * b_per_w

        def body(j, carry):
            b = base + j * chunk
            pltpu.sync_copy(idx_hbm.at[pl.ds(b, chunk)], idx_v)
            pltpu.async_copy(tbl_hbm.at[idx_v], rows_v, sem).wait()
            pltpu.sync_copy(rows_v, out_hbm.at[pl.ds(b, chunk)])
            return carry

        lax.fori_loop(0, nchunk, body, 0)

    return k


def _gather_rows(tbl, idxf):
    return _make_sc_gather()(tbl, idxf)


_ATAN_C = [0.99997726, -0.33262347, 0.19354346, -0.11643287, 0.05265332,
           -0.01172120]


def _atan_core(t):
    s = t * t
    r = jnp.float32(_ATAN_C[5])
    for c in _ATAN_C[4::-1]:
        r = r * s + jnp.float32(c)
    return t * r


def _atan2(y, x):
    ax = jnp.abs(x)
    ay = jnp.abs(y)
    hi = jnp.maximum(ax, ay)
    lo = jnp.minimum(ax, ay)
    t = lo / jnp.maximum(hi, jnp.float32(1e-30))
    a = _atan_core(t)
    a = jnp.where(ay > ax, jnp.float32(np.pi / 2) - a, a)
    a = jnp.where(x < 0.0, jnp.float32(np.pi) - a, a)
    a = jnp.where(y < 0.0, -a, a)
    return jnp.where(hi == 0.0, 0.0, a)


def _mlp1_body(g_ref, c_ref, w0_ref, b0_ref, z_ref, s_ref):
    g = g_ref[...]
    cq = c_ref[...]
    crep = jnp.broadcast_to(cq[:, None, :], (QMB, NS, 3)).reshape(MB, 3)
    cn = g[:, 64:67] - crep
    sub = MB // 128
    x = cn[:, 0:1].reshape(sub, 128)
    y = cn[:, 1:2].reshape(sub, 128)
    z = cn[:, 2:3].reshape(sub, 128)
    rho = jnp.sqrt(x * x + y * y + z * z)
    safe = jnp.where(rho == 0.0, 1.0, rho)
    u = jnp.clip(z / safe, -1.0, 1.0)
    theta = _atan2(jnp.sqrt(jnp.maximum(1.0 - u * u, 0.0)), u)
    theta = jnp.where(rho == 0.0, 0.0, theta) * jnp.float32(1.0 / np.pi)
    phi = _atan2(y, x) * jnp.float32(1.0 / (2.0 * np.pi)) + 0.5
    geo = jnp.concatenate(
        [cn, rho.reshape(MB, 1), theta.reshape(MB, 1), phi.reshape(MB, 1),
         g[:, 67:70]], axis=1)
    w0g = w0_ref[:, 0:9]
    z1 = g[:, 0:64] + lax.dot_general(
        geo, w0g, (((1,), (1,)), ((), ())),
        preferred_element_type=jnp.float32) + b0_ref[...]
    z_ref[...] = z1

    @pl.when(pl.program_id(0) == 0)
    def _():
        s_ref[...] = jnp.zeros_like(s_ref)

    upd = jnp.concatenate(
        [jnp.sum(z1, axis=0, keepdims=True),
         jnp.sum(z1 * z1, axis=0, keepdims=True),
         jnp.zeros((6, 64), jnp.float32)], axis=0)
    s_ref[...] += upd


def _mlp1(gathered, center, w0, b0r):
    return pl.pallas_call(
        _mlp1_body,
        grid=(M // MB,),
        in_specs=[
            pl.BlockSpec((MB, TBL_D), lambda i: (i, 0)),
            pl.BlockSpec((QMB, 3), lambda i: (i, 0)),
            pl.BlockSpec((64, 137), lambda i: (0, 0)),
            pl.BlockSpec((1, 64), lambda i: (0, 0)),
        ],
        out_specs=(
            pl.BlockSpec((MB, 64), lambda i: (i, 0)),
            pl.BlockSpec((8, 64), lambda i: (0, 0)),
        ),
        out_shape=(
            jax.ShapeDtypeStruct((M, 64), jnp.float32),
            jax.ShapeDtypeStruct((8, 64), jnp.float32),
        ),
    )(gathered, center, w0, b0r)


def _mid_body(cin, cout, z_ref, s_ref, gam_ref, bet_ref, w_ref, b_ref,
              zo_ref, so_ref):
    s = s_ref[...]
    mean = s[0:1, :] / CNT
    var = s[1:2, :] / CNT - mean * mean
    scale = gam_ref[...] * lax.rsqrt(var + EPS)
    shift = bet_ref[...] - mean * scale
    h = jnp.maximum(z_ref[...] * scale + shift, 0.0)
    z2 = lax.dot_general(h, w_ref[...], (((1,), (1,)), ((), ())),
                         preferred_element_type=jnp.float32) + b_ref[...]
    zo_ref[...] = z2

    @pl.when(pl.program_id(0) == 0)
    def _():
        so_ref[...] = jnp.zeros_like(so_ref)

    upd = jnp.concatenate(
        [jnp.sum(z2, axis=0, keepdims=True),
         jnp.sum(z2 * z2, axis=0, keepdims=True),
         jnp.zeros((6, cout), jnp.float32)], axis=0)
    so_ref[...] += upd


def _mid(cin, cout, z, s, gam, bet, w, b):
    return pl.pallas_call(
        functools.partial(_mid_body, cin, cout),
        grid=(M // MB,),
        in_specs=[
            pl.BlockSpec((MB, cin), lambda i: (i, 0)),
            pl.BlockSpec((8, cin), lambda i: (0, 0)),
            pl.BlockSpec((1, cin), lambda i: (0, 0)),
            pl.BlockSpec((1, cin), lambda i: (0, 0)),
            pl.BlockSpec((cout, cin), lambda i: (0, 0)),
            pl.BlockSpec((1, cout), lambda i: (0, 0)),
        ],
        out_specs=(
            pl.BlockSpec((MB, cout), lambda i: (i, 0)),
            pl.BlockSpec((8, cout), lambda i: (0, 0)),
        ),
        out_shape=(
            jax.ShapeDtypeStruct((M, cout), jnp.float32),
            jax.ShapeDtypeStruct((8, cout), jnp.float32),
        ),
    )(z, s, gam, bet, w, b)


def _fin_body(z_ref, s_ref, gam_ref, bet_ref, o_ref):
    s = s_ref[...]
    mean = s[0:1, :] / CNT
    var = s[1:2, :] / CNT - mean * mean
    scale = gam_ref[...] * lax.rsqrt(var + EPS)
    shift = bet_ref[...] - mean * scale
    h = jnp.maximum(z_ref[...] * scale + shift, 0.0)
    o_ref[...] = jnp.max(h.reshape(QMB, NS, 128), axis=1)


def _fin(z, s, gam, bet):
    return pl.pallas_call(
        _fin_body,
        grid=(M // MB,),
        in_specs=[
            pl.BlockSpec((MB, 128), lambda i: (i, 0)),
            pl.BlockSpec((8, 128), lambda i: (0, 0)),
            pl.BlockSpec((1, 128), lambda i: (0, 0)),
            pl.BlockSpec((1, 128), lambda i: (0, 0)),
        ],
        out_specs=pl.BlockSpec((QMB, 128), lambda i: (i, 0)),
        out_shape=jax.ShapeDtypeStruct((N, 128), jnp.float32),
    )(z, s, gam, bet)


def kernel(center, normal, feature, offset, W0, b0, gamma0, beta0,
           W1, b1, gamma1, beta1, W2, b2, gamma2, beta2):
    cpad = jnp.pad(center, ((0, NPAD - N), (0, 0)), constant_values=1e4)
    psqc, tbl = _prep(cpad, normal, feature, W0)
    idx = _knn(cpad, cpad, psqc.reshape(1, NPAD))
    idxf = idx[:N].reshape(-1)
    gathered = _gather_rows(tbl, idxf)
    z1, s1 = _mlp1(gathered, center, W0, b0.reshape(1, 64))
    z2, s2 = _mid(64, 64, z1, s1, gamma0.reshape(1, 64), beta0.reshape(1, 64),
                  W1, b1.reshape(1, 64))
    z3, s3 = _mid(64, 128, z2, s2, gamma1.reshape(1, 64), beta1.reshape(1, 64),
                  W2, b2.reshape(1, 128))
    nf = _fin(z3, s3, gamma2.reshape(1, 128), beta2.reshape(1, 128))
    return (center, normal, nf, offset)


# speedup vs baseline: 5.6471x; 1.4391x over previous
import functools
import numpy as np
import jax
import jax.numpy as jnp
from jax import lax
from jax.experimental import pallas as pl
from jax.experimental.pallas import tpu as pltpu
from jax.experimental.pallas import tpu_sc as plsc

N = 10000
NS = 32
NPAD = 10240
NQPAD = 10112
QB = 128
MB = 6400
QMB = 200
M = N * NS
TBL_D = 128
BIG = float(1e30)
BIGI = int(2**30)
CNT = float(M)
EPS = float(1e-5)


def _prep_body(cpad_ref, nrm_ref, feat_ref, w0_ref, psqc_ref, tbl_ref):
    c = cpad_ref[...]
    x, y, z = c[:, 0:1], c[:, 1:2], c[:, 2:3]
    psqc_ref[...] = (x * x + y * y) + z * z
    w0f = w0_ref[:, 9:137]
    pf = lax.dot_general(feat_ref[...], w0f, (((1,), (1,)), ((), ())),
                         preferred_element_type=jnp.float32)
    tbl_ref[...] = jnp.concatenate(
        [pf, c[:N], nrm_ref[...], jnp.zeros((N, TBL_D - 70), jnp.float32)],
        axis=1)


def _prep(cpad, normal, feature, w0):
    return pl.pallas_call(
        _prep_body,
        out_shape=(
            jax.ShapeDtypeStruct((NPAD, 1), jnp.float32),
            jax.ShapeDtypeStruct((N, TBL_D), jnp.float32),
        ),
    )(cpad, normal, feature, w0)


def _knn_body(qp_ref, pp_ref, psqr_ref, idx_ref):
    q = qp_ref[...]
    p = pp_ref[...]
    qx, qy, qz = q[:, 0:1], q[:, 1:2], q[:, 2:3]
    qsq = (qx * qx + qy * qy) + qz * qz
    dot2 = lax.dot_general(q.astype(jnp.bfloat16), p.astype(jnp.bfloat16),
                           (((1,), (1,)), ((), ())),
                           preferred_element_type=jnp.float32)
    d = qsq + psqr_ref[0:1, :] - 2.0 * dot2
    d3 = d.reshape(QB, 10, 1024)
    i10 = lax.broadcasted_iota(jnp.int32, (QB, 10, 1024), 1)
    i1024 = lax.broadcasted_iota(jnp.int32, (QB, 10, 1024), 2)
    col3 = i10 * 1024 + i1024
    vals, idxs = [], []
    for _ in range(3):
        m = jnp.min(d3, axis=1)
        t = jnp.where(d3 == m[:, None, :], col3, BIGI)
        ai = jnp.min(t, axis=1)
        d3 = jnp.where(t == ai[:, None, :], BIG, d3)
        vals.append(m)
        idxs.append(ai)
    v4 = jnp.stack(vals, axis=1)
    c4 = jnp.stack(idxs, axis=1)
    v2 = v4.reshape(QB, 24, 128)
    c2 = c4.reshape(QB, 24, 128)
    vals2, idxs2 = [], []
    for _ in range(6):
        m = jnp.min(v2, axis=1)
        t = jnp.where(v2 == m[:, None, :], c2, BIGI)
        ai = jnp.min(t, axis=1)
        v2 = jnp.where(t == ai[:, None, :], BIG, v2)
        vals2.append(m)
        idxs2.append(ai)
    cand = jnp.stack(vals2, axis=1).reshape(QB, 768)
    cidx = jnp.stack(idxs2, axis=1).reshape(QB, 768)
    cols = []
    for _ in range(NS):
        m = jnp.min(cand, axis=1)
        t = jnp.where(cand == m[:, None], cidx, BIGI)
        ai = jnp.min(t, axis=1)
        cand = jnp.where(t == ai[:, None], BIG, cand)
        cols.append(ai)
    idx_ref[...] = jnp.stack(cols, axis=1)


def _knn(qp, pp, psqr):
    return pl.pallas_call(
        _knn_body,
        grid=(NQPAD // QB,),
        in_specs=[
            pl.BlockSpec((QB, 3), lambda i: (i, 0)),
            pl.BlockSpec((NPAD, 3), lambda i: (0, 0)),
            pl.BlockSpec((1, NPAD), lambda i: (0, 0)),
        ],
        out_specs=pl.BlockSpec((QB, NS), lambda i: (i, 0)),
        out_shape=jax.ShapeDtypeStruct((NQPAD, NS), jnp.int32),
    )(qp, pp, psqr)


def _make_sc_gather():
    info = plsc.get_sparse_core_info()
    nw = info.num_cores * info.num_subcores
    b_per_w = M // nw
    chunk = 400
    nchunk = b_per_w // chunk
    mesh = plsc.VectorSubcoreMesh(core_axis_name="c", subcore_axis_name="s")

    @functools.partial(
        pl.kernel, mesh=mesh,
        out_type=jax.ShapeDtypeStruct((M, TBL_D), jnp.float32),
        scratch_types=[
            pltpu.VMEM((chunk,), jnp.int32),
            pltpu.VMEM((chunk, TBL_D), jnp.float32),
            pltpu.SemaphoreType.DMA,
        ],
    )
    def k(tbl_hbm, idx_hbm, out_hbm, idx_v, rows_v, sem):
        wid = lax.axis_index("s") * info.num_cores + lax.axis_index("c")
        base = wid * b_per_w

        def body(j, carry):
            b = base + j * chunk
            pltpu.sync_copy(idx_hbm.at[pl.ds(b, chunk)], idx_v)
            pltpu.async_copy(tbl_hbm.at[idx_v], rows_v, sem).wait()
            pltpu.sync_copy(rows_v, out_hbm.at[pl.ds(b, chunk)])
            return carry

        lax.fori_loop(0, nchunk, body, 0)

    return k


def _gather_rows(tbl, idxf):
    return _make_sc_gather()(tbl, idxf)


_ATAN_C = [0.99997726, -0.33262347, 0.19354346, -0.11643287, 0.05265332,
           -0.01172120]


def _atan_core(t):
    s = t * t
    r = jnp.float32(_ATAN_C[5])
    for c in _ATAN_C[4::-1]:
        r = r * s + jnp.float32(c)
    return t * r


def _atan2(y, x):
    ax = jnp.abs(x)
    ay = jnp.abs(y)
    hi = jnp.maximum(ax, ay)
    lo = jnp.minimum(ax, ay)
    t = lo / jnp.maximum(hi, jnp.float32(1e-30))
    a = _atan_core(t)
    a = jnp.where(ay > ax, jnp.float32(np.pi / 2) - a, a)
    a = jnp.where(x < 0.0, jnp.float32(np.pi) - a, a)
    a = jnp.where(y < 0.0, -a, a)
    return jnp.where(hi == 0.0, 0.0, a)


def _mlp1_body(g_ref, c_ref, w0_ref, b0_ref, z_ref, s_ref):
    g = g_ref[...]
    cq = c_ref[...]
    crep = jnp.broadcast_to(cq[:, None, :], (QMB, NS, 3)).reshape(MB, 3)
    cn = g[:, 64:67] - crep
    sub = MB // 128
    x = cn[:, 0:1].reshape(sub, 128)
    y = cn[:, 1:2].reshape(sub, 128)
    z = cn[:, 2:3].reshape(sub, 128)
    rho = jnp.sqrt(x * x + y * y + z * z)
    safe = jnp.where(rho == 0.0, 1.0, rho)
    u = jnp.clip(z / safe, -1.0, 1.0)
    theta = _atan2(jnp.sqrt(jnp.maximum(1.0 - u * u, 0.0)), u)
    theta = jnp.where(rho == 0.0, 0.0, theta) * jnp.float32(1.0 / np.pi)
    phi = _atan2(y, x) * jnp.float32(1.0 / (2.0 * np.pi)) + 0.5
    geo = jnp.concatenate(
        [cn, rho.reshape(MB, 1), theta.reshape(MB, 1), phi.reshape(MB, 1),
         g[:, 67:70]], axis=1)
    w0g = w0_ref[:, 0:9]
    z1 = g[:, 0:64] + lax.dot_general(
        geo, w0g, (((1,), (1,)), ((), ())),
        preferred_element_type=jnp.float32) + b0_ref[...]
    z_ref[...] = z1

    @pl.when(pl.program_id(0) == 0)
    def _():
        s_ref[...] = jnp.zeros_like(s_ref)

    upd = jnp.concatenate(
        [jnp.sum(z1, axis=0, keepdims=True),
         jnp.sum(z1 * z1, axis=0, keepdims=True),
         jnp.zeros((6, 64), jnp.float32)], axis=0)
    s_ref[...] += upd


def _mlp1(gathered, center, w0, b0r):
    return pl.pallas_call(
        _mlp1_body,
        grid=(M // MB,),
        in_specs=[
            pl.BlockSpec((MB, TBL_D), lambda i: (i, 0)),
            pl.BlockSpec((QMB, 3), lambda i: (i, 0)),
            pl.BlockSpec((64, 137), lambda i: (0, 0)),
            pl.BlockSpec((1, 64), lambda i: (0, 0)),
        ],
        out_specs=(
            pl.BlockSpec((MB, 64), lambda i: (i, 0)),
            pl.BlockSpec((8, 64), lambda i: (0, 0)),
        ),
        out_shape=(
            jax.ShapeDtypeStruct((M, 64), jnp.float32),
            jax.ShapeDtypeStruct((8, 64), jnp.float32),
        ),
    )(gathered, center, w0, b0r)


def _mid_body(cin, cout, z_ref, s_ref, gam_ref, bet_ref, w_ref, b_ref,
              zo_ref, so_ref):
    s = s_ref[...]
    mean = s[0:1, :] / CNT
    var = s[1:2, :] / CNT - mean * mean
    scale = gam_ref[...] * lax.rsqrt(var + EPS)
    shift = bet_ref[...] - mean * scale
    h = jnp.maximum(z_ref[...] * scale + shift, 0.0)
    z2 = lax.dot_general(h, w_ref[...], (((1,), (1,)), ((), ())),
                         preferred_element_type=jnp.float32) + b_ref[...]
    zo_ref[...] = z2

    @pl.when(pl.program_id(0) == 0)
    def _():
        so_ref[...] = jnp.zeros_like(so_ref)

    upd = jnp.concatenate(
        [jnp.sum(z2, axis=0, keepdims=True),
         jnp.sum(z2 * z2, axis=0, keepdims=True),
         jnp.zeros((6, cout), jnp.float32)], axis=0)
    so_ref[...] += upd


def _mid(cin, cout, z, s, gam, bet, w, b):
    return pl.pallas_call(
        functools.partial(_mid_body, cin, cout),
        grid=(M // MB,),
        in_specs=[
            pl.BlockSpec((MB, cin), lambda i: (i, 0)),
            pl.BlockSpec((8, cin), lambda i: (0, 0)),
            pl.BlockSpec((1, cin), lambda i: (0, 0)),
            pl.BlockSpec((1, cin), lambda i: (0, 0)),
            pl.BlockSpec((cout, cin), lambda i: (0, 0)),
            pl.BlockSpec((1, cout), lambda i: (0, 0)),
        ],
        out_specs=(
            pl.BlockSpec((MB, cout), lambda i: (i, 0)),
            pl.BlockSpec((8, cout), lambda i: (0, 0)),
        ),
        out_shape=(
            jax.ShapeDtypeStruct((M, cout), jnp.float32),
            jax.ShapeDtypeStruct((8, cout), jnp.float32),
        ),
    )(z, s, gam, bet, w, b)


def _fin_body(z_ref, s_ref, gam_ref, bet_ref, o_ref):
    s = s_ref[...]
    mean = s[0:1, :] / CNT
    var = s[1:2, :] / CNT - mean * mean
    scale = gam_ref[...] * lax.rsqrt(var + EPS)
    shift = bet_ref[...] - mean * scale
    h = jnp.maximum(z_ref[...] * scale + shift, 0.0)
    o_ref[...] = jnp.max(h.reshape(QMB, NS, 128), axis=1)


def _fin(z, s, gam, bet):
    return pl.pallas_call(
        _fin_body,
        grid=(M // MB,),
        in_specs=[
            pl.BlockSpec((MB, 128), lambda i: (i, 0)),
            pl.BlockSpec((8, 128), lambda i: (0, 0)),
            pl.BlockSpec((1, 128), lambda i: (0, 0)),
            pl.BlockSpec((1, 128), lambda i: (0, 0)),
        ],
        out_specs=pl.BlockSpec((QMB, 128), lambda i: (i, 0)),
        out_shape=jax.ShapeDtypeStruct((N, 128), jnp.float32),
    )(z, s, gam, bet)


def kernel(center, normal, feature, offset, W0, b0, gamma0, beta0,
           W1, b1, gamma1, beta1, W2, b2, gamma2, beta2):
    cpad = jnp.pad(center, ((0, NPAD - N), (0, 0)), constant_values=1e4)
    psqc, tbl = _prep(cpad, normal, feature, W0)
    idx = _knn(cpad, cpad, psqc.reshape(1, NPAD))
    idxf = idx[:N].reshape(-1)
    gathered = _gather_rows(tbl, idxf)
    z1, s1 = _mlp1(gathered, center, W0, b0.reshape(1, 64))
    z2, s2 = _mid(64, 64, z1, s1, gamma0.reshape(1, 64), beta0.reshape(1, 64),
                  W1, b1.reshape(1, 64))
    z3, s3 = _mid(64, 128, z2, s2, gamma1.reshape(1, 64), beta1.reshape(1, 64),
                  W2, b2.reshape(1, 128))
    nf = _fin(z3, s3, gamma2.reshape(1, 128), beta2.reshape(1, 128))
    return (center, normal, nf, offset)


# speedup vs baseline: 5.7250x; 1.0138x over previous
import functools
import numpy as np
import jax
import jax.numpy as jnp
from jax import lax
from jax.experimental import pallas as pl
from jax.experimental.pallas import tpu as pltpu
from jax.experimental.pallas import tpu_sc as plsc

N = 10000
NS = 32
NPAD = 10240
NQPAD = 10112
QB = 128
MB = 6400
QMB = 200
M = N * NS
TBL_D = 128
BIG = float(1e30)
BIGI = int(2**30)
CNT = float(M)
EPS = float(1e-5)


def _prep_body(cpad_ref, nrm_ref, feat_ref, w0_ref, psqc_ref, tbl_ref):
    c = cpad_ref[...]
    x, y, z = c[:, 0:1], c[:, 1:2], c[:, 2:3]
    psqc_ref[...] = (x * x + y * y) + z * z
    w0f = w0_ref[:, 9:137]
    pf = lax.dot_general(feat_ref[...], w0f, (((1,), (1,)), ((), ())),
                         preferred_element_type=jnp.float32)
    tbl_ref[...] = jnp.concatenate(
        [pf, c[:N], nrm_ref[...], jnp.zeros((N, TBL_D - 70), jnp.float32)],
        axis=1)


def _prep(cpad, normal, feature, w0):
    return pl.pallas_call(
        _prep_body,
        out_shape=(
            jax.ShapeDtypeStruct((NPAD, 1), jnp.float32),
            jax.ShapeDtypeStruct((N, TBL_D), jnp.float32),
        ),
    )(cpad, normal, feature, w0)


def _knn_body(qp_ref, pp_ref, psqr_ref, idx_ref):
    q = qp_ref[...]
    p = pp_ref[...]
    qx, qy, qz = q[:, 0:1], q[:, 1:2], q[:, 2:3]
    qsq = (qx * qx + qy * qy) + qz * qz
    dot2 = lax.dot_general(q.astype(jnp.bfloat16), p.astype(jnp.bfloat16),
                           (((1,), (1,)), ((), ())),
                           preferred_element_type=jnp.float32)
    d = qsq + psqr_ref[0:1, :] - 2.0 * dot2
    d3 = d.reshape(QB, 10, 1024)
    i10 = lax.broadcasted_iota(jnp.int32, (QB, 10, 1024), 1)
    i1024 = lax.broadcasted_iota(jnp.int32, (QB, 10, 1024), 2)
    col3 = i10 * 1024 + i1024
    vals, idxs = [], []
    for _ in range(3):
        m = jnp.min(d3, axis=1)
        t = jnp.where(d3 == m[:, None, :], col3, BIGI)
        ai = jnp.min(t, axis=1)
        d3 = jnp.where(t == ai[:, None, :], BIG, d3)
        vals.append(m)
        idxs.append(ai)
    v4 = jnp.stack(vals, axis=1)
    c4 = jnp.stack(idxs, axis=1)
    v2 = v4.reshape(QB, 24, 128)
    c2 = c4.reshape(QB, 24, 128)
    vals2, idxs2 = [], []
    for _ in range(5):
        m = jnp.min(v2, axis=1)
        t = jnp.where(v2 == m[:, None, :], c2, BIGI)
        ai = jnp.min(t, axis=1)
        v2 = jnp.where(t == ai[:, None, :], BIG, v2)
        vals2.append(m)
        idxs2.append(ai)
    cand = jnp.stack(vals2, axis=1).reshape(QB, 640)
    cidx = jnp.stack(idxs2, axis=1).reshape(QB, 640)
    cols = []
    for _ in range(NS):
        m = jnp.min(cand, axis=1)
        t = jnp.where(cand == m[:, None], cidx, BIGI)
        ai = jnp.min(t, axis=1)
        cand = jnp.where(t == ai[:, None], BIG, cand)
        cols.append(ai)
    idx_ref[...] = jnp.stack(cols, axis=1)


def _knn(qp, pp, psqr):
    return pl.pallas_call(
        _knn_body,
        grid=(NQPAD // QB,),
        in_specs=[
            pl.BlockSpec((QB, 3), lambda i: (i, 0)),
            pl.BlockSpec((NPAD, 3), lambda i: (0, 0)),
            pl.BlockSpec((1, NPAD), lambda i: (0, 0)),
        ],
        out_specs=pl.BlockSpec((QB, NS), lambda i: (i, 0)),
        out_shape=jax.ShapeDtypeStruct((NQPAD, NS), jnp.int32),
    )(qp, pp, psqr)


def _make_sc_gather():
    info = plsc.get_sparse_core_info()
    nw = info.num_cores * info.num_subcores
    b_per_w = M // nw
    chunk = 400
    nchunk = b_per_w // chunk
    mesh = plsc.VectorSubcoreMesh(core_axis_name="c", subcore_axis_name="s")

    @functools.partial(
        pl.kernel, mesh=mesh,
        out_type=jax.ShapeDtypeStruct((M, TBL_D), jnp.float32),
        scratch_types=[
            pltpu.VMEM((chunk,), jnp.int32),
            pltpu.VMEM((chunk, TBL_D), jnp.float32),
            pltpu.SemaphoreType.DMA,
        ],
    )
    def k(tbl_hbm, idx_hbm, out_hbm, idx_v, rows_v, sem):
        wid = lax.axis_index("s") * info.num_cores + lax.axis_index("c")
        base = wid * b_per_w

        def body(j, carry):
            b = base + j * chunk
            pltpu.sync_copy(idx_hbm.at[pl.ds(b, chunk)], idx_v)
            pltpu.async_copy(tbl_hbm.at[idx_v], rows_v, sem).wait()
            pltpu.sync_copy(rows_v, out_hbm.at[pl.ds(b, chunk)])
            return carry

        lax.fori_loop(0, nchunk, body, 0)

    return k


def _gather_rows(tbl, idxf):
    return _make_sc_gather()(tbl, idxf)


_ATAN_C = [0.99997726, -0.33262347, 0.19354346, -0.11643287, 0.05265332,
           -0.01172120]


def _atan_core(t):
    s = t * t
    r = jnp.float32(_ATAN_C[5])
    for c in _ATAN_C[4::-1]:
        r = r * s + jnp.float32(c)
    return t * r


def _atan2(y, x):
    ax = jnp.abs(x)
    ay = jnp.abs(y)
    hi = jnp.maximum(ax, ay)
    lo = jnp.minimum(ax, ay)
    t = lo / jnp.maximum(hi, jnp.float32(1e-30))
    a = _atan_core(t)
    a = jnp.where(ay > ax, jnp.float32(np.pi / 2) - a, a)
    a = jnp.where(x < 0.0, jnp.float32(np.pi) - a, a)
    a = jnp.where(y < 0.0, -a, a)
    return jnp.where(hi == 0.0, 0.0, a)


def _mlp1_body(g_ref, c_ref, w0_ref, b0_ref, z_ref, s_ref):
    g = g_ref[...]
    cq = c_ref[...]
    crep = jnp.broadcast_to(cq[:, None, :], (QMB, NS, 3)).reshape(MB, 3)
    cn = g[:, 64:67] - crep
    sub = MB // 128
    x = cn[:, 0:1].reshape(sub, 128)
    y = cn[:, 1:2].reshape(sub, 128)
    z = cn[:, 2:3].reshape(sub, 128)
    rho = jnp.sqrt(x * x + y * y + z * z)
    safe = jnp.where(rho == 0.0, 1.0, rho)
    u = jnp.clip(z / safe, -1.0, 1.0)
    theta = _atan2(jnp.sqrt(jnp.maximum(1.0 - u * u, 0.0)), u)
    theta = jnp.where(rho == 0.0, 0.0, theta) * jnp.float32(1.0 / np.pi)
    phi = _atan2(y, x) * jnp.float32(1.0 / (2.0 * np.pi)) + 0.5
    geo = jnp.concatenate(
        [cn, rho.reshape(MB, 1), theta.reshape(MB, 1), phi.reshape(MB, 1),
         g[:, 67:70]], axis=1)
    w0g = w0_ref[:, 0:9]
    z1 = g[:, 0:64] + lax.dot_general(
        geo, w0g, (((1,), (1,)), ((), ())),
        preferred_element_type=jnp.float32) + b0_ref[...]
    z_ref[...] = z1

    @pl.when(pl.program_id(0) == 0)
    def _():
        s_ref[...] = jnp.zeros_like(s_ref)

    upd = jnp.concatenate(
        [jnp.sum(z1, axis=0, keepdims=True),
         jnp.sum(z1 * z1, axis=0, keepdims=True),
         jnp.zeros((6, 64), jnp.float32)], axis=0)
    s_ref[...] += upd


def _mlp1(gathered, center, w0, b0r):
    return pl.pallas_call(
        _mlp1_body,
        grid=(M // MB,),
        in_specs=[
            pl.BlockSpec((MB, TBL_D), lambda i: (i, 0)),
            pl.BlockSpec((QMB, 3), lambda i: (i, 0)),
            pl.BlockSpec((64, 137), lambda i: (0, 0)),
            pl.BlockSpec((1, 64), lambda i: (0, 0)),
        ],
        out_specs=(
            pl.BlockSpec((MB, 64), lambda i: (i, 0)),
            pl.BlockSpec((8, 64), lambda i: (0, 0)),
        ),
        out_shape=(
            jax.ShapeDtypeStruct((M, 64), jnp.float32),
            jax.ShapeDtypeStruct((8, 64), jnp.float32),
        ),
    )(gathered, center, w0, b0r)


def _mid_body(cin, cout, z_ref, s_ref, gam_ref, bet_ref, w_ref, b_ref,
              zo_ref, so_ref):
    s = s_ref[...]
    mean = s[0:1, :] / CNT
    var = s[1:2, :] / CNT - mean * mean
    scale = gam_ref[...] * lax.rsqrt(var + EPS)
    shift = bet_ref[...] - mean * scale
    h = jnp.maximum(z_ref[...] * scale + shift, 0.0)
    z2 = lax.dot_general(h, w_ref[...], (((1,), (1,)), ((), ())),
                         preferred_element_type=jnp.float32) + b_ref[...]
    zo_ref[...] = z2

    @pl.when(pl.program_id(0) == 0)
    def _():
        so_ref[...] = jnp.zeros_like(so_ref)

    upd = jnp.concatenate(
        [jnp.sum(z2, axis=0, keepdims=True),
         jnp.sum(z2 * z2, axis=0, keepdims=True),
         jnp.zeros((6, cout), jnp.float32)], axis=0)
    so_ref[...] += upd


def _mid(cin, cout, z, s, gam, bet, w, b):
    return pl.pallas_call(
        functools.partial(_mid_body, cin, cout),
        grid=(M // MB,),
        in_specs=[
            pl.BlockSpec((MB, cin), lambda i: (i, 0)),
            pl.BlockSpec((8, cin), lambda i: (0, 0)),
            pl.BlockSpec((1, cin), lambda i: (0, 0)),
            pl.BlockSpec((1, cin), lambda i: (0, 0)),
            pl.BlockSpec((cout, cin), lambda i: (0, 0)),
            pl.BlockSpec((1, cout), lambda i: (0, 0)),
        ],
        out_specs=(
            pl.BlockSpec((MB, cout), lambda i: (i, 0)),
            pl.BlockSpec((8, cout), lambda i: (0, 0)),
        ),
        out_shape=(
            jax.ShapeDtypeStruct((M, cout), jnp.float32),
            jax.ShapeDtypeStruct((8, cout), jnp.float32),
        ),
    )(z, s, gam, bet, w, b)


def _fin_body(z_ref, s_ref, gam_ref, bet_ref, o_ref):
    s = s_ref[...]
    mean = s[0:1, :] / CNT
    var = s[1:2, :] / CNT - mean * mean
    scale = gam_ref[...] * lax.rsqrt(var + EPS)
    shift = bet_ref[...] - mean * scale
    h = jnp.maximum(z_ref[...] * scale + shift, 0.0)
    o_ref[...] = jnp.max(h.reshape(QMB, NS, 128), axis=1)


def _fin(z, s, gam, bet):
    return pl.pallas_call(
        _fin_body,
        grid=(M // MB,),
        in_specs=[
            pl.BlockSpec((MB, 128), lambda i: (i, 0)),
            pl.BlockSpec((8, 128), lambda i: (0, 0)),
            pl.BlockSpec((1, 128), lambda i: (0, 0)),
            pl.BlockSpec((1, 128), lambda i: (0, 0)),
        ],
        out_specs=pl.BlockSpec((QMB, 128), lambda i: (i, 0)),
        out_shape=jax.ShapeDtypeStruct((N, 128), jnp.float32),
    )(z, s, gam, bet)


def kernel(center, normal, feature, offset, W0, b0, gamma0, beta0,
           W1, b1, gamma1, beta1, W2, b2, gamma2, beta2):
    cpad = jnp.pad(center, ((0, NPAD - N), (0, 0)), constant_values=1e4)
    psqc, tbl = _prep(cpad, normal, feature, W0)
    idx = _knn(cpad, cpad, psqc.reshape(1, NPAD))
    idxf = idx[:N].reshape(-1)
    gathered = _gather_rows(tbl, idxf)
    z1, s1 = _mlp1(gathered, center, W0, b0.reshape(1, 64))
    z2, s2 = _mid(64, 64, z1, s1, gamma0.reshape(1, 64), beta0.reshape(1, 64),
                  W1, b1.reshape(1, 64))
    z3, s3 = _mid(64, 128, z2, s2, gamma1.reshape(1, 64), beta1.reshape(1, 64),
                  W2, b2.reshape(1, 128))
    nf = _fin(z3, s3, gamma2.reshape(1, 128), beta2.reshape(1, 128))
    return (center, normal, nf, offset)
